# Initial kernel scaffold; baseline (speedup 1.0000x reference)
import jax
import jax.numpy as jnp
from jax.experimental import pallas as pl


def kernel(input, table):
    raise NotImplementedError("write your pallas kernel here")


# speedup vs baseline: 2.9653x; 2.9653x over previous
import functools

import jax
import jax.numpy as jnp
from jax import lax
from jax.experimental import pallas as pl
from jax.experimental.pallas import tpu as pltpu
from jax.experimental.pallas import tpu_sc as plsc

TOTAL_LOOKUPS = 4096 * 50
EMBED = 128
NUM_CORES = 2
NUM_SUBCORES = 16
NUM_WORKERS = NUM_CORES * NUM_SUBCORES
PER_WORKER = TOTAL_LOOKUPS // NUM_WORKERS
CHUNK = 128
N_CHUNKS = PER_WORKER // CHUNK


def _make_emb_kernel():
  mesh = plsc.VectorSubcoreMesh(core_axis_name="c", subcore_axis_name="s")

  @functools.partial(
      pl.kernel,
      mesh=mesh,
      out_type=jax.ShapeDtypeStruct((TOTAL_LOOKUPS, EMBED), jnp.float32),
      scratch_types=[
          pltpu.VMEM((N_CHUNKS, CHUNK), jnp.int32),
          pltpu.VMEM((CHUNK, EMBED), jnp.float32),
          pltpu.SemaphoreType.DMA,
      ],
  )
  def emb(idx_hbm, table_hbm, out_hbm, idx_v, rows_v, gsem):
    wid = lax.axis_index("s") * NUM_CORES + lax.axis_index("c")
    base = wid * PER_WORKER
    pltpu.sync_copy(idx_hbm.at[wid], idx_v)

    def body(g, carry):
      pltpu.async_copy(table_hbm.at[idx_v.at[g]], rows_v, gsem).wait()
      pltpu.sync_copy(rows_v, out_hbm.at[pl.ds(base + g * CHUNK, CHUNK)])
      return carry

    lax.fori_loop(0, N_CHUNKS, body, 0)

  return emb


_EMB = _make_emb_kernel()


@jax.jit
def kernel(input, table):
  idx = input.reshape(NUM_WORKERS, N_CHUNKS, CHUNK)
  out = _EMB(idx, table)
  return out.reshape(input.shape[0], input.shape[1], EMBED)


# speedup vs baseline: 3.3355x; 1.1249x over previous
import functools

import jax
import jax.numpy as jnp
from jax import lax
from jax.experimental import pallas as pl
from jax.experimental.pallas import tpu as pltpu
from jax.experimental.pallas import tpu_sc as plsc

TOTAL_LOOKUPS = 4096 * 50
EMBED = 128
NUM_CORES = 2
NUM_SUBCORES = 16
NUM_WORKERS = NUM_CORES * NUM_SUBCORES
PER_WORKER = TOTAL_LOOKUPS // NUM_WORKERS
CHUNK = 128
N_CHUNKS = PER_WORKER // CHUNK


def _make_emb_kernel():
  mesh = plsc.VectorSubcoreMesh(core_axis_name="c", subcore_axis_name="s")

  @functools.partial(
      pl.kernel,
      mesh=mesh,
      out_type=jax.ShapeDtypeStruct((TOTAL_LOOKUPS, EMBED), jnp.float32),
      scratch_types=[
          pltpu.VMEM((N_CHUNKS, CHUNK), jnp.int32),
          pltpu.VMEM((2, CHUNK, EMBED), jnp.float32),
          pltpu.SemaphoreType.DMA,
          pltpu.SemaphoreType.DMA,
      ],
  )
  def emb(idx_hbm, table_hbm, out_hbm, idx_v, rows_v, gsem, osem):
    wid = lax.axis_index("s") * NUM_CORES + lax.axis_index("c")
    base = wid * PER_WORKER
    pltpu.sync_copy(idx_hbm.at[wid], idx_v)

    pltpu.async_copy(table_hbm.at[idx_v.at[0]], rows_v.at[0], gsem)

    def body(g, carry):
      cur = lax.rem(g, 2)
      nxt = 1 - cur

      @pl.when(g > 0)
      def _wait_prev_store():
        pltpu.make_async_copy(
            rows_v.at[nxt],
            out_hbm.at[pl.ds(base + (g - 1) * CHUNK, CHUNK)],
            osem,
        ).wait()

      @pl.when(g + 1 < N_CHUNKS)
      def _fire_next_gather():
        pltpu.async_copy(table_hbm.at[idx_v.at[g + 1]], rows_v.at[nxt], gsem)

      pltpu.make_async_copy(
          table_hbm.at[idx_v.at[g]], rows_v.at[cur], gsem).wait()
      pltpu.async_copy(
          rows_v.at[cur], out_hbm.at[pl.ds(base + g * CHUNK, CHUNK)], osem)
      return carry

    lax.fori_loop(0, N_CHUNKS, body, 0)
    pltpu.make_async_copy(
        rows_v.at[(N_CHUNKS - 1) % 2],
        out_hbm.at[pl.ds(base + (N_CHUNKS - 1) * CHUNK, CHUNK)],
        osem,
    ).wait()

  return emb


_EMB = _make_emb_kernel()


@jax.jit
def kernel(input, table):
  idx = input.reshape(NUM_WORKERS, N_CHUNKS, CHUNK)
  out = _EMB(idx, table)
  return out.reshape(input.shape[0], input.shape[1], EMBED)


# speedup vs baseline: 3.3568x; 1.0064x over previous
import functools

import jax
import jax.numpy as jnp
from jax import lax
from jax.experimental import pallas as pl
from jax.experimental.pallas import tpu as pltpu
from jax.experimental.pallas import tpu_sc as plsc

TOTAL_LOOKUPS = 4096 * 50
EMBED = 128
NUM_CORES = 2
NUM_SUBCORES = 16
NUM_WORKERS = NUM_CORES * NUM_SUBCORES
PER_WORKER = TOTAL_LOOKUPS // NUM_WORKERS
CHUNK = 128
N_CHUNKS = PER_WORKER // CHUNK
NBUF = 4


def _make_emb_kernel():
  mesh = plsc.VectorSubcoreMesh(core_axis_name="c", subcore_axis_name="s")

  @functools.partial(
      pl.kernel,
      mesh=mesh,
      out_type=jax.ShapeDtypeStruct((TOTAL_LOOKUPS, EMBED), jnp.float32),
      scratch_types=[
          pltpu.VMEM((N_CHUNKS, CHUNK), jnp.int32),
          pltpu.VMEM((NBUF, CHUNK, EMBED), jnp.float32),
          pltpu.SemaphoreType.DMA,
          pltpu.SemaphoreType.DMA,
      ],
  )
  def emb(idx_hbm, table_hbm, out_hbm, idx_v, rows_v, gsem, osem):
    wid = lax.axis_index("s") * NUM_CORES + lax.axis_index("c")
    base = wid * PER_WORKER
    pltpu.sync_copy(idx_hbm.at[wid], idx_v)

    for p in range(NBUF - 1):
      pltpu.async_copy(table_hbm.at[idx_v.at[p]], rows_v.at[p], gsem)

    def body(g, carry):
      cur = lax.rem(g, NBUF)

      @pl.when(g > 0)
      def _wait_prev_store():
        pltpu.make_async_copy(
            rows_v.at[lax.rem(g + NBUF - 1, NBUF)],
            out_hbm.at[pl.ds(base + (g - 1) * CHUNK, CHUNK)],
            osem,
        ).wait()

      @pl.when(g + NBUF - 1 < N_CHUNKS)
      def _fire_next_gather():
        pltpu.async_copy(
            table_hbm.at[idx_v.at[g + NBUF - 1]],
            rows_v.at[lax.rem(g + NBUF - 1, NBUF)], gsem)

      pltpu.make_async_copy(
          table_hbm.at[idx_v.at[g]], rows_v.at[cur], gsem).wait()
      pltpu.async_copy(
          rows_v.at[cur], out_hbm.at[pl.ds(base + g * CHUNK, CHUNK)], osem)
      return carry

    lax.fori_loop(0, N_CHUNKS, body, 0)
    pltpu.make_async_copy(
        rows_v.at[(N_CHUNKS - 1) % NBUF],
        out_hbm.at[pl.ds(base + (N_CHUNKS - 1) * CHUNK, CHUNK)],
        osem,
    ).wait()

  return emb


_EMB = _make_emb_kernel()


@jax.jit
def kernel(input, table):
  idx = input.reshape(NUM_WORKERS, N_CHUNKS, CHUNK)
  out = _EMB(idx, table)
  return out.reshape(input.shape[0], input.shape[1], EMBED)


# speedup vs baseline: 5.9750x; 1.7800x over previous
import functools

import jax
import jax.numpy as jnp
from jax import lax
from jax.experimental import pallas as pl
from jax.experimental.pallas import tpu as pltpu
from jax.experimental.pallas import tpu_sc as plsc

BATCH = 4096
HIST = 50
EMBED = 128
NUM_CORES = 2
NUM_SUBCORES = 16
NUM_WORKERS = NUM_CORES * NUM_SUBCORES
ROWS_PER_W = BATCH // NUM_WORKERS
NBUF = 4


def _make_emb_kernel():
  mesh = plsc.VectorSubcoreMesh(core_axis_name="c", subcore_axis_name="s")

  @functools.partial(
      pl.kernel,
      mesh=mesh,
      out_type=jax.ShapeDtypeStruct((BATCH, HIST, EMBED), jnp.float32),
      scratch_types=[
          pltpu.VMEM((ROWS_PER_W, HIST), jnp.int32),
          pltpu.VMEM((NBUF, HIST, EMBED), jnp.float32),
          pltpu.SemaphoreType.DMA,
          pltpu.SemaphoreType.DMA,
      ],
  )
  def emb(idx_hbm, table_hbm, out_hbm, idx_v, rows_v, gsem, osem):
    wid = lax.axis_index("s") * NUM_CORES + lax.axis_index("c")
    row0 = wid * ROWS_PER_W
    pltpu.sync_copy(idx_hbm.at[pl.ds(row0, ROWS_PER_W)], idx_v)

    for p in range(NBUF - 1):
      pltpu.async_copy(table_hbm.at[idx_v.at[p]], rows_v.at[p], gsem)

    def body(g, carry):
      cur = lax.rem(g, NBUF)

      @pl.when(g > 0)
      def _wait_prev_store():
        pltpu.make_async_copy(
            rows_v.at[lax.rem(g + NBUF - 1, NBUF)],
            out_hbm.at[row0 + g - 1],
            osem,
        ).wait()

      @pl.when(g + NBUF - 1 < ROWS_PER_W)
      def _fire_next_gather():
        pltpu.async_copy(
            table_hbm.at[idx_v.at[g + NBUF - 1]],
            rows_v.at[lax.rem(g + NBUF - 1, NBUF)], gsem)

      pltpu.make_async_copy(
          table_hbm.at[idx_v.at[g]], rows_v.at[cur], gsem).wait()
      pltpu.async_copy(rows_v.at[cur], out_hbm.at[row0 + g], osem)
      return carry

    lax.fori_loop(0, ROWS_PER_W, body, 0)
    pltpu.make_async_copy(
        rows_v.at[(ROWS_PER_W - 1) % NBUF],
        out_hbm.at[row0 + ROWS_PER_W - 1],
        osem,
    ).wait()

  return emb


_EMB = _make_emb_kernel()


@jax.jit
def kernel(input, table):
  return _EMB(input, table)


# speedup vs baseline: 5.9774x; 1.0004x over previous
import functools

import jax
import jax.numpy as jnp
from jax import lax
from jax.experimental import pallas as pl
from jax.experimental.pallas import tpu as pltpu
from jax.experimental.pallas import tpu_sc as plsc

BATCH = 4096
HIST = 50
EMBED = 128
NUM_CORES = 2
NUM_SUBCORES = 16
NUM_WORKERS = NUM_CORES * NUM_SUBCORES
ROWS_PER_W = BATCH // NUM_WORKERS
NBUF = 4


def _make_emb_kernel():
  mesh = plsc.VectorSubcoreMesh(core_axis_name="c", subcore_axis_name="s")

  @functools.partial(
      pl.kernel,
      mesh=mesh,
      compiler_params=pltpu.CompilerParams(use_tc_tiling_on_sc=True),
      out_type=jax.ShapeDtypeStruct((BATCH, HIST, EMBED), jnp.float32),
      scratch_types=[
          pltpu.VMEM((ROWS_PER_W, HIST), jnp.int32),
          pltpu.VMEM((NBUF, HIST, EMBED), jnp.float32),
          pltpu.SemaphoreType.DMA,
          pltpu.SemaphoreType.DMA,
      ],
  )
  def emb(idx_hbm, table_hbm, out_hbm, idx_v, rows_v, gsem, osem):
    wid = lax.axis_index("s") * NUM_CORES + lax.axis_index("c")
    row0 = wid * ROWS_PER_W
    pltpu.sync_copy(idx_hbm.at[pl.ds(row0, ROWS_PER_W)], idx_v)

    for p in range(NBUF - 1):
      pltpu.async_copy(table_hbm.at[idx_v.at[p]], rows_v.at[p], gsem)

    def body(g, carry):
      cur = lax.rem(g, NBUF)

      @pl.when(g > 0)
      def _wait_prev_store():
        pltpu.make_async_copy(
            rows_v.at[lax.rem(g + NBUF - 1, NBUF)],
            out_hbm.at[row0 + g - 1],
            osem,
        ).wait()

      @pl.when(g + NBUF - 1 < ROWS_PER_W)
      def _fire_next_gather():
        pltpu.async_copy(
            table_hbm.at[idx_v.at[g + NBUF - 1]],
            rows_v.at[lax.rem(g + NBUF - 1, NBUF)], gsem)

      pltpu.make_async_copy(
          table_hbm.at[idx_v.at[g]], rows_v.at[cur], gsem).wait()
      pltpu.async_copy(rows_v.at[cur], out_hbm.at[row0 + g], osem)
      return carry

    lax.fori_loop(0, ROWS_PER_W, body, 0)
    pltpu.make_async_copy(
        rows_v.at[(ROWS_PER_W - 1) % NBUF],
        out_hbm.at[row0 + ROWS_PER_W - 1],
        osem,
    ).wait()

  return emb


_EMB = _make_emb_kernel()


@jax.jit
def kernel(input, table):
  return _EMB(input, table)


# speedup vs baseline: 10.8083x; 1.8082x over previous
import functools

import jax
import jax.numpy as jnp
from jax import lax
from jax.experimental import pallas as pl
from jax.experimental.pallas import tpu as pltpu
from jax.experimental.pallas import tpu_sc as plsc

BATCH = 4096
HIST = 50
EMBED = 128
NUM_CORES = 2
NUM_SUBCORES = 16
NUM_WORKERS = NUM_CORES * NUM_SUBCORES
COLS_PER_W = BATCH // NUM_WORKERS
NBUF = 4


def _make_emb_kernel():
  mesh = plsc.VectorSubcoreMesh(core_axis_name="c", subcore_axis_name="s")

  @functools.partial(
      pl.kernel,
      mesh=mesh,
      out_type=jax.ShapeDtypeStruct((HIST, BATCH, EMBED), jnp.float32),
      scratch_types=[
          pltpu.VMEM((HIST, COLS_PER_W), jnp.int32),
          pltpu.VMEM((NBUF, COLS_PER_W, EMBED), jnp.float32),
          pltpu.SemaphoreType.DMA,
          pltpu.SemaphoreType.DMA,
      ],
  )
  def emb(idx_hbm, table_hbm, out_hbm, idx_v, rows_v, gsem, osem):
    wid = lax.axis_index("s") * NUM_CORES + lax.axis_index("c")
    col0 = wid * COLS_PER_W
    pltpu.sync_copy(idx_hbm.at[:, pl.ds(col0, COLS_PER_W)], idx_v)

    for p in range(NBUF - 1):
      pltpu.async_copy(table_hbm.at[idx_v.at[p]], rows_v.at[p], gsem)

    def body(g, carry):
      cur = lax.rem(g, NBUF)

      @pl.when(g > 0)
      def _wait_prev_store():
        pltpu.make_async_copy(
            rows_v.at[lax.rem(g + NBUF - 1, NBUF)],
            out_hbm.at[g - 1].at[pl.ds(col0, COLS_PER_W)],
            osem,
        ).wait()

      @pl.when(g + NBUF - 1 < HIST)
      def _fire_next_gather():
        pltpu.async_copy(
            table_hbm.at[idx_v.at[g + NBUF - 1]],
            rows_v.at[lax.rem(g + NBUF - 1, NBUF)], gsem)

      pltpu.make_async_copy(
          table_hbm.at[idx_v.at[g]], rows_v.at[cur], gsem).wait()
      pltpu.async_copy(
          rows_v.at[cur], out_hbm.at[g].at[pl.ds(col0, COLS_PER_W)], osem)
      return carry

    lax.fori_loop(0, HIST, body, 0)
    pltpu.make_async_copy(
        rows_v.at[(HIST - 1) % NBUF],
        out_hbm.at[HIST - 1].at[pl.ds(col0, COLS_PER_W)],
        osem,
    ).wait()

  return emb


_EMB = _make_emb_kernel()


@jax.jit
def kernel(input, table):
  out_t = _EMB(input.T, table)
  return out_t.transpose(1, 0, 2)
